# baseline (speedup 1.0000x reference)
import jax
import jax.numpy as jnp
from jax.experimental import pallas as pl
from jax.experimental.pallas import tpu as pltpu

_B, _A, _T, _D = 256, 128, 80, 2
_ROW = _A * _T * _D
_BLOCK_B = 8


def _body(ts_ref, ac_ref, o_ref, n_ref, out_ref):
    i = pl.program_id(0)
    for j in range(_BLOCK_B):
        t = ts_ref[i * _BLOCK_B + j]
        ac = ac_ref[t]
        sa = jnp.sqrt(ac)
        sb = jnp.sqrt(1.0 - ac)
        out_ref[j, :] = sa * o_ref[j, :] + sb * n_ref[j, :]


def kernel(original_samples, noise, timesteps, speed_labels, steer_labels, agents_interested, alphas_cumprod):
    del speed_labels, steer_labels, agents_interested
    o2 = original_samples.reshape(_B, _ROW)
    n2 = noise.reshape(_B, _ROW)
    out = pl.pallas_call(
        _body,
        grid=(_B // _BLOCK_B,),
        in_specs=[
            pl.BlockSpec(memory_space=pltpu.SMEM),
            pl.BlockSpec(memory_space=pltpu.SMEM),
            pl.BlockSpec((_BLOCK_B, _ROW), lambda i: (i, 0)),
            pl.BlockSpec((_BLOCK_B, _ROW), lambda i: (i, 0)),
        ],
        out_specs=pl.BlockSpec((_BLOCK_B, _ROW), lambda i: (i, 0)),
        out_shape=jax.ShapeDtypeStruct((_B, _ROW), jnp.float32),
    )(timesteps, alphas_cumprod, o2, n2)
    return out.reshape(_B, _A, _T, _D)


# speedup vs baseline: 1.5417x; 1.5417x over previous
import jax
import jax.numpy as jnp
from jax.experimental import pallas as pl
from jax.experimental.pallas import tpu as pltpu

_B, _A, _T, _D = 256, 128, 80, 2
_TD = _T * _D
_BLOCK_B = 8


def _body(ts_ref, ac_ref, o_ref, n_ref, out_ref):
    i = pl.program_id(0)
    for j in range(_BLOCK_B):
        t = ts_ref[i * _BLOCK_B + j]
        ac = ac_ref[t]
        sa = jnp.sqrt(ac)
        sb = jnp.sqrt(1.0 - ac)
        out_ref[j, :, :] = sa * o_ref[j, :, :] + sb * n_ref[j, :, :]


def kernel(original_samples, noise, timesteps, speed_labels, steer_labels, agents_interested, alphas_cumprod):
    del speed_labels, steer_labels, agents_interested
    o2 = original_samples.reshape(_B, _A, _TD)
    n2 = noise.reshape(_B, _A, _TD)
    out = pl.pallas_call(
        _body,
        grid=(_B // _BLOCK_B,),
        in_specs=[
            pl.BlockSpec(memory_space=pltpu.SMEM),
            pl.BlockSpec(memory_space=pltpu.SMEM),
            pl.BlockSpec((_BLOCK_B, _A, _TD), lambda i: (i, 0, 0)),
            pl.BlockSpec((_BLOCK_B, _A, _TD), lambda i: (i, 0, 0)),
        ],
        out_specs=pl.BlockSpec((_BLOCK_B, _A, _TD), lambda i: (i, 0, 0)),
        out_shape=jax.ShapeDtypeStruct((_B, _A, _TD), jnp.float32),
    )(timesteps, alphas_cumprod, o2, n2)
    return out.reshape(_B, _A, _T, _D)


# speedup vs baseline: 1.6306x; 1.0577x over previous
import jax
import jax.numpy as jnp
from jax.experimental import pallas as pl
from jax.experimental.pallas import tpu as pltpu

_B, _A, _T, _D = 256, 128, 80, 2
_TD = _T * _D
_BLOCK_B = 32


def _body(ts_ref, ac_ref, o_ref, n_ref, out_ref):
    i = pl.program_id(0)
    for j in range(_BLOCK_B):
        t = ts_ref[i * _BLOCK_B + j]
        ac = ac_ref[t]
        sa = jnp.sqrt(ac)
        sb = jnp.sqrt(1.0 - ac)
        out_ref[j, :, :] = sa * o_ref[j, :, :] + sb * n_ref[j, :, :]


def kernel(original_samples, noise, timesteps, speed_labels, steer_labels, agents_interested, alphas_cumprod):
    del speed_labels, steer_labels, agents_interested
    o2 = original_samples.reshape(_B, _A, _TD)
    n2 = noise.reshape(_B, _A, _TD)
    out = pl.pallas_call(
        _body,
        grid=(_B // _BLOCK_B,),
        in_specs=[
            pl.BlockSpec(memory_space=pltpu.SMEM),
            pl.BlockSpec(memory_space=pltpu.SMEM),
            pl.BlockSpec((_BLOCK_B, _A, _TD), lambda i: (i, 0, 0)),
            pl.BlockSpec((_BLOCK_B, _A, _TD), lambda i: (i, 0, 0)),
        ],
        out_specs=pl.BlockSpec((_BLOCK_B, _A, _TD), lambda i: (i, 0, 0)),
        out_shape=jax.ShapeDtypeStruct((_B, _A, _TD), jnp.float32),
    )(timesteps, alphas_cumprod, o2, n2)
    return out.reshape(_B, _A, _T, _D)


# speedup vs baseline: 7.3036x; 4.4790x over previous
import jax
import jax.numpy as jnp
from jax.experimental import pallas as pl
from jax.experimental.pallas import tpu as pltpu

_B, _A, _T, _D = 256, 128, 80, 2
_TD = _T * _D
_BLOCK_B = 8


def _body(ts_ref, ac_ref, o_ref, n_ref, out_ref):
    i = pl.program_id(0)
    for j in range(_BLOCK_B):
        t = ts_ref[i * _BLOCK_B + j]
        ac = ac_ref[t]
        sa = jnp.sqrt(ac)
        sb = jnp.sqrt(1.0 - ac)
        out_ref[j, :, :] = sa * o_ref[j, :, :] + sb * n_ref[j, :, :]


def kernel(original_samples, noise, timesteps, speed_labels, steer_labels, agents_interested, alphas_cumprod):
    del speed_labels, steer_labels, agents_interested
    o2 = jnp.transpose(original_samples, (0, 2, 3, 1)).reshape(_B, _TD, _A)
    n2 = jnp.transpose(noise, (0, 2, 3, 1)).reshape(_B, _TD, _A)
    out = pl.pallas_call(
        _body,
        grid=(_B // _BLOCK_B,),
        in_specs=[
            pl.BlockSpec(memory_space=pltpu.SMEM),
            pl.BlockSpec(memory_space=pltpu.SMEM),
            pl.BlockSpec((_BLOCK_B, _TD, _A), lambda i: (i, 0, 0)),
            pl.BlockSpec((_BLOCK_B, _TD, _A), lambda i: (i, 0, 0)),
        ],
        out_specs=pl.BlockSpec((_BLOCK_B, _TD, _A), lambda i: (i, 0, 0)),
        out_shape=jax.ShapeDtypeStruct((_B, _TD, _A), jnp.float32),
    )(timesteps, alphas_cumprod, o2, n2)
    return jnp.transpose(out.reshape(_B, _T, _D, _A), (0, 3, 1, 2))


# speedup vs baseline: 10.1096x; 1.3842x over previous
import jax
import jax.numpy as jnp
from jax.experimental import pallas as pl
from jax.experimental.pallas import tpu as pltpu

_B, _A, _T, _D = 256, 128, 80, 2
_TD = _T * _D
_BLOCK_B = 16


def _body(ts_ref, ac_ref, o_ref, n_ref, out_ref):
    i = pl.program_id(0)
    for j in range(_BLOCK_B):
        t = ts_ref[i * _BLOCK_B + j]
        ac = ac_ref[t]
        sa = jnp.sqrt(ac)
        sb = jnp.sqrt(1.0 - ac)
        out_ref[j, :, :] = sa * o_ref[j, :, :] + sb * n_ref[j, :, :]


def kernel(original_samples, noise, timesteps, speed_labels, steer_labels, agents_interested, alphas_cumprod):
    del speed_labels, steer_labels, agents_interested
    o2 = jnp.transpose(original_samples, (0, 2, 3, 1)).reshape(_B, _TD, _A)
    n2 = jnp.transpose(noise, (0, 2, 3, 1)).reshape(_B, _TD, _A)
    out = pl.pallas_call(
        _body,
        grid=(_B // _BLOCK_B,),
        in_specs=[
            pl.BlockSpec(memory_space=pltpu.SMEM),
            pl.BlockSpec(memory_space=pltpu.SMEM),
            pl.BlockSpec((_BLOCK_B, _TD, _A), lambda i: (i, 0, 0)),
            pl.BlockSpec((_BLOCK_B, _TD, _A), lambda i: (i, 0, 0)),
        ],
        out_specs=pl.BlockSpec((_BLOCK_B, _TD, _A), lambda i: (i, 0, 0)),
        out_shape=jax.ShapeDtypeStruct((_B, _TD, _A), jnp.float32),
    )(timesteps, alphas_cumprod, o2, n2)
    return jnp.transpose(out.reshape(_B, _T, _D, _A), (0, 3, 1, 2))


# speedup vs baseline: 11.3873x; 1.1264x over previous
import jax
import jax.numpy as jnp
from jax.experimental import pallas as pl
from jax.experimental.pallas import tpu as pltpu

_B, _A, _T, _D = 256, 128, 80, 2
_TD = _T * _D
_BLOCK_B = 64


def _body(ts_ref, ac_ref, o_ref, n_ref, out_ref):
    i = pl.program_id(0)
    for j in range(_BLOCK_B):
        t = ts_ref[i * _BLOCK_B + j]
        ac = ac_ref[t]
        sa = jnp.sqrt(ac)
        sb = jnp.sqrt(1.0 - ac)
        out_ref[j, :, :] = sa * o_ref[j, :, :] + sb * n_ref[j, :, :]


def kernel(original_samples, noise, timesteps, speed_labels, steer_labels, agents_interested, alphas_cumprod):
    del speed_labels, steer_labels, agents_interested
    o2 = jnp.transpose(original_samples, (0, 2, 3, 1)).reshape(_B, _TD, _A)
    n2 = jnp.transpose(noise, (0, 2, 3, 1)).reshape(_B, _TD, _A)
    out = pl.pallas_call(
        _body,
        grid=(_B // _BLOCK_B,),
        in_specs=[
            pl.BlockSpec(memory_space=pltpu.SMEM),
            pl.BlockSpec(memory_space=pltpu.SMEM),
            pl.BlockSpec((_BLOCK_B, _TD, _A), lambda i: (i, 0, 0)),
            pl.BlockSpec((_BLOCK_B, _TD, _A), lambda i: (i, 0, 0)),
        ],
        out_specs=pl.BlockSpec((_BLOCK_B, _TD, _A), lambda i: (i, 0, 0)),
        out_shape=jax.ShapeDtypeStruct((_B, _TD, _A), jnp.float32),
    )(timesteps, alphas_cumprod, o2, n2)
    return jnp.transpose(out.reshape(_B, _T, _D, _A), (0, 3, 1, 2))
